# baseline (speedup 1.0000x reference)
import functools
import math

import jax
import jax.numpy as jnp
from jax.experimental import pallas as pl
from jax.experimental.pallas import tpu as pltpu

_H = 768
_NH = 12
_DH = 64
_FFN = 3072
_FFN_CHUNK = 512
_SCALE = 1.0 / math.sqrt(_DH)
_VMEM_LIMIT = 48 * 1024 * 1024


def _ln(x, g, b, eps=1e-12):
    mu = jnp.mean(x, axis=-1, keepdims=True)
    d = x - mu
    var = jnp.mean(d * d, axis=-1, keepdims=True)
    return d * jax.lax.rsqrt(var + eps) * g + b


def _gelu(y):
    c = math.sqrt(2.0 / math.pi)
    return 0.5 * y * (1.0 + jnp.tanh(c * (y + 0.044715 * y * y * y)))


def _layer_body(has_in_ln, *refs):
    if has_in_ln:
        (x_ref, m_ref, qkvw_ref, qkvb_ref, ow_ref, ob_ref, sag_ref, sab_ref,
         f1w_ref, f1b_ref, f2w_ref, f2b_ref, og_ref, obeta_ref, ig_ref, ib_ref,
         o_ref, q_sc, k_sc, v_sc, ctx_sc) = refs
    else:
        (x_ref, m_ref, qkvw_ref, qkvb_ref, ow_ref, ob_ref, sag_ref, sab_ref,
         f1w_ref, f1b_ref, f2w_ref, f2b_ref, og_ref, obeta_ref,
         o_ref, q_sc, k_sc, v_sc, ctx_sc) = refs

    xv = x_ref[0].astype(jnp.float32)
    if has_in_ln:
        xv = _ln(xv, ig_ref[...], ib_ref[...])
    x_bf = xv.astype(jnp.bfloat16)
    resid0 = x_bf.astype(jnp.float32)

    def dot_k2(lhs, w_ref_, col=None):
        hk = lhs.shape[1] // 2
        sl = slice(None) if col is None else col
        acc = jnp.dot(lhs[:, :hk], w_ref_[:hk, sl],
                      preferred_element_type=jnp.float32)
        return acc + jnp.dot(lhs[:, hk:], w_ref_[hk:, sl],
                             preferred_element_type=jnp.float32)

    for off, sc in ((0, q_sc), (_H, k_sc), (2 * _H, v_sc)):
        col = slice(off, off + _H)
        sc[...] = (dot_k2(x_bf, qkvw_ref, col=col)
                   + qkvb_ref[:, col]).astype(jnp.bfloat16)

    mrow = m_ref[0]
    ones_k = jnp.ones((x_ref.shape[1], 128), jnp.bfloat16)
    for h in range(_NH):
        q_h = q_sc[:, h * _DH:(h + 1) * _DH]
        k_h = k_sc[:, h * _DH:(h + 1) * _DH]
        v_h = v_sc[:, h * _DH:(h + 1) * _DH]
        s = jax.lax.dot_general(q_h, k_h, (((1,), (1,)), ((), ())),
                                preferred_element_type=jnp.float32)
        s = jnp.where(mrow > 0.0, s, jnp.float32(-1e9))
        m = jnp.max(s, axis=-1, keepdims=True)
        p = (jnp.exp(s - m)).astype(jnp.bfloat16)
        l = jnp.dot(p, ones_k, preferred_element_type=jnp.float32)[:, :1]
        ctx = jnp.dot(p, v_h, preferred_element_type=jnp.float32)
        ctx = ctx * pl.reciprocal(l, approx=True)
        ctx_sc[:, h * _DH:(h + 1) * _DH] = ctx.astype(jnp.bfloat16)

    attn = dot_k2(ctx_sc[...], ow_ref)
    attn = (attn + ob_ref[...]).astype(jnp.bfloat16).astype(jnp.float32)
    x1 = _ln(attn + resid0, sag_ref[...], sab_ref[...]).astype(jnp.bfloat16)

    h2 = None
    for c in range(0, _FFN, _FFN_CHUNK):
        h1 = dot_k2(x1, f1w_ref, col=slice(c, c + _FFN_CHUNK))
        h1 = _gelu(h1 + f1b_ref[:, c:c + _FFN_CHUNK]).astype(jnp.bfloat16)
        d = jnp.dot(h1, f2w_ref[c:c + _FFN_CHUNK, :],
                    preferred_element_type=jnp.float32)
        h2 = d if h2 is None else h2 + d
    h2 = (h2 + f2b_ref[...]).astype(jnp.bfloat16).astype(jnp.float32)
    x2 = _ln(h2 + x1.astype(jnp.float32), og_ref[...], obeta_ref[...])
    o_ref[0] = x2.astype(jnp.bfloat16)


def _encoder_layer(x, mask3, qkv_w, qkv_b, o_w, o_b, sa_g, sa_b,
                   f1w, f1b, f2w, f2b, out_g, out_b, in_ln=None):
    B, S, H = x.shape

    def full(shape):
        return pl.BlockSpec(shape, lambda b: (0,) * len(shape))

    in_specs = [
        pl.BlockSpec((1, S, H), lambda b: (b, 0, 0)),
        pl.BlockSpec((1, 1, S), lambda b: (b, 0, 0)),
        full((H, 3 * H)), full((1, 3 * H)),
        full((H, H)), full((1, H)),
        full((1, H)), full((1, H)),
        full((H, _FFN)), full((1, _FFN)),
        full((_FFN, H)), full((1, H)),
        full((1, H)), full((1, H)),
    ]
    args = [x, mask3, qkv_w, qkv_b, o_w, o_b, sa_g, sa_b,
            f1w, f1b, f2w, f2b, out_g, out_b]
    if in_ln is not None:
        in_specs += [full((1, H)), full((1, H))]
        args += list(in_ln)

    return pl.pallas_call(
        functools.partial(_layer_body, in_ln is not None),
        grid=(B,),
        in_specs=in_specs,
        out_specs=pl.BlockSpec((1, S, H), lambda b: (b, 0, 0)),
        out_shape=jax.ShapeDtypeStruct((B, S, H), jnp.bfloat16),
        scratch_shapes=[pltpu.VMEM((S, H), jnp.bfloat16),
                        pltpu.VMEM((S, H), jnp.bfloat16),
                        pltpu.VMEM((S, H), jnp.bfloat16),
                        pltpu.VMEM((S, H), jnp.bfloat16)],
        compiler_params=pltpu.CompilerParams(
            dimension_semantics=("parallel",),
            vmem_limit_bytes=_VMEM_LIMIT),
    )(*args)


def _row(v):
    return v.reshape(1, -1).astype(jnp.float32)


def kernel(input_ids, attention_mask, word_emb, pos_emb, emb_ln_g, emb_ln_b, cls_w, cls_b,
           layer0_q_w, layer0_q_b, layer0_k_w, layer0_k_b, layer0_v_w, layer0_v_b,
           layer0_o_w, layer0_o_b, layer0_sa_ln_g, layer0_sa_ln_b,
           layer0_ffn1_w, layer0_ffn1_b, layer0_ffn2_w, layer0_ffn2_b,
           layer0_out_ln_g, layer0_out_ln_b,
           layer1_q_w, layer1_q_b, layer1_k_w, layer1_k_b, layer1_v_w, layer1_v_b,
           layer1_o_w, layer1_o_b, layer1_sa_ln_g, layer1_sa_ln_b,
           layer1_ffn1_w, layer1_ffn1_b, layer1_ffn2_w, layer1_ffn2_b,
           layer1_out_ln_g, layer1_out_ln_b,
           layer2_q_w, layer2_q_b, layer2_k_w, layer2_k_b, layer2_v_w, layer2_v_b,
           layer2_o_w, layer2_o_b, layer2_sa_ln_g, layer2_sa_ln_b,
           layer2_ffn1_w, layer2_ffn1_b, layer2_ffn2_w, layer2_ffn2_b,
           layer2_out_ln_g, layer2_out_ln_b,
           layer3_q_w, layer3_q_b, layer3_k_w, layer3_k_b, layer3_v_w, layer3_v_b,
           layer3_o_w, layer3_o_b, layer3_sa_ln_g, layer3_sa_ln_b,
           layer3_ffn1_w, layer3_ffn1_b, layer3_ffn2_w, layer3_ffn2_b,
           layer3_out_ln_g, layer3_out_ln_b):
    B, S = input_ids.shape

    layers = [
        (layer0_q_w, layer0_q_b, layer0_k_w, layer0_k_b, layer0_v_w, layer0_v_b,
         layer0_o_w, layer0_o_b, layer0_sa_ln_g, layer0_sa_ln_b,
         layer0_ffn1_w, layer0_ffn1_b, layer0_ffn2_w, layer0_ffn2_b,
         layer0_out_ln_g, layer0_out_ln_b),
        (layer1_q_w, layer1_q_b, layer1_k_w, layer1_k_b, layer1_v_w, layer1_v_b,
         layer1_o_w, layer1_o_b, layer1_sa_ln_g, layer1_sa_ln_b,
         layer1_ffn1_w, layer1_ffn1_b, layer1_ffn2_w, layer1_ffn2_b,
         layer1_out_ln_g, layer1_out_ln_b),
        (layer2_q_w, layer2_q_b, layer2_k_w, layer2_k_b, layer2_v_w, layer2_v_b,
         layer2_o_w, layer2_o_b, layer2_sa_ln_g, layer2_sa_ln_b,
         layer2_ffn1_w, layer2_ffn1_b, layer2_ffn2_w, layer2_ffn2_b,
         layer2_out_ln_g, layer2_out_ln_b),
        (layer3_q_w, layer3_q_b, layer3_k_w, layer3_k_b, layer3_v_w, layer3_v_b,
         layer3_o_w, layer3_o_b, layer3_sa_ln_g, layer3_sa_ln_b,
         layer3_ffn1_w, layer3_ffn1_b, layer3_ffn2_w, layer3_ffn2_b,
         layer3_out_ln_g, layer3_out_ln_b),
    ]

    x = (word_emb[input_ids] + pos_emb[:S][None, :, :]).astype(jnp.bfloat16)
    mask3 = attention_mask.astype(jnp.float32).reshape(B, 1, S)

    for li, (q_w, q_b, k_w, k_b, v_w, v_b, o_w, o_b, sa_g, sa_b,
             f1w, f1b, f2w, f2b, out_g, out_b) in enumerate(layers):
        qkv_w = jnp.concatenate([q_w * _SCALE, k_w, v_w],
                                axis=1).astype(jnp.bfloat16)
        qkv_b = jnp.concatenate([q_b * _SCALE, k_b, v_b]).reshape(1, -1)
        in_ln = (_row(emb_ln_g), _row(emb_ln_b)) if li == 0 else None
        x = _encoder_layer(
            x, mask3, qkv_w, qkv_b,
            o_w.astype(jnp.bfloat16), _row(o_b), _row(sa_g), _row(sa_b),
            f1w.astype(jnp.bfloat16), _row(f1b),
            f2w.astype(jnp.bfloat16), _row(f2b), _row(out_g), _row(out_b),
            in_ln=in_ln)

    pooled = x[:, 0, :].astype(jnp.float32)
    return pooled @ cls_w + cls_b


# speedup vs baseline: 1.2633x; 1.2633x over previous
import functools
import math

import jax
import jax.numpy as jnp
from jax.experimental import pallas as pl
from jax.experimental.pallas import tpu as pltpu

_H = 768
_NH = 12
_DH = 64
_FFN = 3072
_FFN_CHUNK = 512
_SCALE = 1.0 / math.sqrt(_DH)
_VMEM_LIMIT = 48 * 1024 * 1024


def _ln(x, g, b, eps=1e-12):
    mu = jnp.mean(x, axis=-1, keepdims=True)
    d = x - mu
    var = jnp.mean(d * d, axis=-1, keepdims=True)
    return d * jax.lax.rsqrt(var + eps) * g + b


def _gelu(y):
    c = math.sqrt(2.0 / math.pi)
    return 0.5 * y * (1.0 + jnp.tanh(c * (y + 0.044715 * y * y * y)))


def _layer_body(has_in_ln, *refs):
    if has_in_ln:
        (x_ref, m_ref, qkvw_ref, qkvb_ref, ow_ref, ob_ref, sag_ref, sab_ref,
         f1w_ref, f1b_ref, f2w_ref, f2b_ref, og_ref, obeta_ref, ig_ref, ib_ref,
         o_ref, qkv_sc, ctx_sc, kt_sc, khat_sc, vhat_sc) = refs
    else:
        (x_ref, m_ref, qkvw_ref, qkvb_ref, ow_ref, ob_ref, sag_ref, sab_ref,
         f1w_ref, f1b_ref, f2w_ref, f2b_ref, og_ref, obeta_ref,
         o_ref, qkv_sc, ctx_sc, kt_sc, khat_sc, vhat_sc) = refs

    TB, S, H = x_ref.shape
    xv = x_ref[...].reshape(TB * S, H).astype(jnp.float32)
    if has_in_ln:
        xv = _ln(xv, ig_ref[...], ib_ref[...])
    x_bf = xv.astype(jnp.bfloat16)
    resid0 = x_bf.astype(jnp.float32)

    def dot_k2(lhs, w_ref_, col=None):
        hk = lhs.shape[1] // 2
        sl = slice(None) if col is None else col
        acc = jnp.dot(lhs[:, :hk], w_ref_[:hk, sl],
                      preferred_element_type=jnp.float32)
        return acc + jnp.dot(lhs[:, hk:], w_ref_[hk:, sl],
                             preferred_element_type=jnp.float32)

    qkv = dot_k2(x_bf, qkvw_ref)
    qkv_sc[...] = (qkv + qkvb_ref[...]).astype(jnp.bfloat16)

    NG = _NH // 4
    @pl.when(pl.program_id(0) == 0)
    def _zero_blockdiag():
        khat_sc[...] = jnp.zeros_like(khat_sc)
        vhat_sc[...] = jnp.zeros_like(vhat_sc)

    kt_sc[...] = qkv_sc[:, _H:2 * _H].T
    for h in range(_NH):
        j, i = divmod(h, 4)
        khat_sc[_DH * i:_DH * (i + 1),
                1024 * j + 256 * i:1024 * j + 256 * i + 256] = (
            kt_sc[_DH * h:_DH * (h + 1), :])
        vhat_sc[S * i:S * (i + 1),
                256 * j + _DH * i:256 * j + _DH * (i + 1)] = (
            qkv_sc[:, 2 * _H + _DH * h:2 * _H + _DH * (h + 1)])

    mrow = m_ref[0]
    keep = mrow > 0.0
    for j in range(NG):
        s4 = jnp.dot(qkv_sc[:, 256 * j:256 * (j + 1)],
                     khat_sc[:, 1024 * j:1024 * (j + 1)],
                     preferred_element_type=jnp.float32)
        ps, linvs = [], []
        for i in range(4):
            s = jnp.where(keep, s4[:, 256 * i:256 * (i + 1)],
                          jnp.float32(-1e9))
            m = jnp.max(s, axis=-1, keepdims=True)
            p = jnp.exp(s - m)
            l = jnp.sum(p, axis=-1, keepdims=True)
            ps.append(p.astype(jnp.bfloat16))
            linvs.append(jnp.broadcast_to(
                pl.reciprocal(l, approx=True), (S, _DH)))
        ctx4 = jnp.dot(jnp.concatenate(ps, axis=1),
                       vhat_sc[:, 256 * j:256 * (j + 1)],
                       preferred_element_type=jnp.float32)
        ctx4 = ctx4 * jnp.concatenate(linvs, axis=1)
        ctx_sc[:, 256 * j:256 * (j + 1)] = ctx4.astype(jnp.bfloat16)

    attn = dot_k2(ctx_sc[...], ow_ref)
    attn = (attn + ob_ref[...]).astype(jnp.bfloat16).astype(jnp.float32)
    x1 = _ln(attn + resid0, sag_ref[...], sab_ref[...]).astype(jnp.bfloat16)

    h2 = None
    for c in range(0, _FFN, _FFN_CHUNK):
        h1 = dot_k2(x1, f1w_ref, col=slice(c, c + _FFN_CHUNK))
        h1 = _gelu(h1 + f1b_ref[:, c:c + _FFN_CHUNK]).astype(jnp.bfloat16)
        d = jnp.dot(h1, f2w_ref[c:c + _FFN_CHUNK, :],
                    preferred_element_type=jnp.float32)
        h2 = d if h2 is None else h2 + d
    h2 = (h2 + f2b_ref[...]).astype(jnp.bfloat16).astype(jnp.float32)
    x2 = _ln(h2 + x1.astype(jnp.float32), og_ref[...], obeta_ref[...])
    o_ref[...] = x2.astype(jnp.bfloat16).reshape(TB, S, H)


def _encoder_layer(x, mask3, qkv_w, qkv_b, o_w, o_b, sa_g, sa_b,
                   f1w, f1b, f2w, f2b, out_g, out_b, in_ln=None, tb=1):
    B, S, H = x.shape

    def full(shape):
        return pl.BlockSpec(shape, lambda b: (0,) * len(shape))

    in_specs = [
        pl.BlockSpec((tb, S, H), lambda b: (b, 0, 0)),
        pl.BlockSpec((tb, 1, S), lambda b: (b, 0, 0)),
        full((H, 3 * H)), full((1, 3 * H)),
        full((H, H)), full((1, H)),
        full((1, H)), full((1, H)),
        full((H, _FFN)), full((1, _FFN)),
        full((_FFN, H)), full((1, H)),
        full((1, H)), full((1, H)),
    ]
    args = [x, mask3, qkv_w, qkv_b, o_w, o_b, sa_g, sa_b,
            f1w, f1b, f2w, f2b, out_g, out_b]
    if in_ln is not None:
        in_specs += [full((1, H)), full((1, H))]
        args += list(in_ln)

    return pl.pallas_call(
        functools.partial(_layer_body, in_ln is not None),
        grid=(B // tb,),
        in_specs=in_specs,
        out_specs=pl.BlockSpec((tb, S, H), lambda b: (b, 0, 0)),
        out_shape=jax.ShapeDtypeStruct((B, S, H), jnp.bfloat16),
        scratch_shapes=[pltpu.VMEM((tb * S, 3 * H), jnp.bfloat16),
                        pltpu.VMEM((tb * S, H), jnp.bfloat16),
                        pltpu.VMEM((H, S), jnp.bfloat16),
                        pltpu.VMEM((S, 3072), jnp.bfloat16),
                        pltpu.VMEM((4 * S, 768), jnp.bfloat16)],
        compiler_params=pltpu.CompilerParams(
            dimension_semantics=("parallel",),
            vmem_limit_bytes=_VMEM_LIMIT),
    )(*args)


def _row(v):
    return v.reshape(1, -1).astype(jnp.float32)


def kernel(input_ids, attention_mask, word_emb, pos_emb, emb_ln_g, emb_ln_b, cls_w, cls_b,
           layer0_q_w, layer0_q_b, layer0_k_w, layer0_k_b, layer0_v_w, layer0_v_b,
           layer0_o_w, layer0_o_b, layer0_sa_ln_g, layer0_sa_ln_b,
           layer0_ffn1_w, layer0_ffn1_b, layer0_ffn2_w, layer0_ffn2_b,
           layer0_out_ln_g, layer0_out_ln_b,
           layer1_q_w, layer1_q_b, layer1_k_w, layer1_k_b, layer1_v_w, layer1_v_b,
           layer1_o_w, layer1_o_b, layer1_sa_ln_g, layer1_sa_ln_b,
           layer1_ffn1_w, layer1_ffn1_b, layer1_ffn2_w, layer1_ffn2_b,
           layer1_out_ln_g, layer1_out_ln_b,
           layer2_q_w, layer2_q_b, layer2_k_w, layer2_k_b, layer2_v_w, layer2_v_b,
           layer2_o_w, layer2_o_b, layer2_sa_ln_g, layer2_sa_ln_b,
           layer2_ffn1_w, layer2_ffn1_b, layer2_ffn2_w, layer2_ffn2_b,
           layer2_out_ln_g, layer2_out_ln_b,
           layer3_q_w, layer3_q_b, layer3_k_w, layer3_k_b, layer3_v_w, layer3_v_b,
           layer3_o_w, layer3_o_b, layer3_sa_ln_g, layer3_sa_ln_b,
           layer3_ffn1_w, layer3_ffn1_b, layer3_ffn2_w, layer3_ffn2_b,
           layer3_out_ln_g, layer3_out_ln_b):
    B, S = input_ids.shape

    layers = [
        (layer0_q_w, layer0_q_b, layer0_k_w, layer0_k_b, layer0_v_w, layer0_v_b,
         layer0_o_w, layer0_o_b, layer0_sa_ln_g, layer0_sa_ln_b,
         layer0_ffn1_w, layer0_ffn1_b, layer0_ffn2_w, layer0_ffn2_b,
         layer0_out_ln_g, layer0_out_ln_b),
        (layer1_q_w, layer1_q_b, layer1_k_w, layer1_k_b, layer1_v_w, layer1_v_b,
         layer1_o_w, layer1_o_b, layer1_sa_ln_g, layer1_sa_ln_b,
         layer1_ffn1_w, layer1_ffn1_b, layer1_ffn2_w, layer1_ffn2_b,
         layer1_out_ln_g, layer1_out_ln_b),
        (layer2_q_w, layer2_q_b, layer2_k_w, layer2_k_b, layer2_v_w, layer2_v_b,
         layer2_o_w, layer2_o_b, layer2_sa_ln_g, layer2_sa_ln_b,
         layer2_ffn1_w, layer2_ffn1_b, layer2_ffn2_w, layer2_ffn2_b,
         layer2_out_ln_g, layer2_out_ln_b),
        (layer3_q_w, layer3_q_b, layer3_k_w, layer3_k_b, layer3_v_w, layer3_v_b,
         layer3_o_w, layer3_o_b, layer3_sa_ln_g, layer3_sa_ln_b,
         layer3_ffn1_w, layer3_ffn1_b, layer3_ffn2_w, layer3_ffn2_b,
         layer3_out_ln_g, layer3_out_ln_b),
    ]

    x = (word_emb[input_ids] + pos_emb[:S][None, :, :]).astype(jnp.bfloat16)
    mask3 = attention_mask.astype(jnp.float32).reshape(B, 1, S)

    for li, (q_w, q_b, k_w, k_b, v_w, v_b, o_w, o_b, sa_g, sa_b,
             f1w, f1b, f2w, f2b, out_g, out_b) in enumerate(layers):
        qkv_w = jnp.concatenate([q_w * _SCALE, k_w, v_w],
                                axis=1).astype(jnp.bfloat16)
        qkv_b = jnp.concatenate([q_b * _SCALE, k_b, v_b]).reshape(1, -1)
        in_ln = (_row(emb_ln_g), _row(emb_ln_b)) if li == 0 else None
        x = _encoder_layer(
            x, mask3, qkv_w, qkv_b,
            o_w.astype(jnp.bfloat16), _row(o_b), _row(sa_g), _row(sa_b),
            f1w.astype(jnp.bfloat16), _row(f1b),
            f2w.astype(jnp.bfloat16), _row(f2b), _row(out_g), _row(out_b),
            in_ln=in_ln)

    pooled = x[:, 0, :].astype(jnp.float32)
    return pooled @ cls_w + cls_b


# speedup vs baseline: 1.3749x; 1.0883x over previous
import functools
import math

import jax
import jax.numpy as jnp
from jax.experimental import pallas as pl
from jax.experimental.pallas import tpu as pltpu

_H = 768
_NH = 12
_DH = 64
_FFN = 3072
_FFN_CHUNK = 512
_SCALE = 1.0 / math.sqrt(_DH)
_VMEM_LIMIT = 48 * 1024 * 1024


def _ln(x, g, b, eps=1e-12):
    mu = jnp.mean(x, axis=-1, keepdims=True)
    d = x - mu
    var = jnp.mean(d * d, axis=-1, keepdims=True)
    return d * jax.lax.rsqrt(var + eps) * g + b


def _gelu(y):
    c = math.sqrt(2.0 / math.pi)
    return 0.5 * y * (1.0 + jnp.tanh(c * (y + 0.044715 * y * y * y)))


def _layer_body(has_in_ln, *refs):
    if has_in_ln:
        (x_ref, m_ref, qkvw_ref, qkvb_ref, ow_ref, ob_ref, sag_ref, sab_ref,
         f1w_ref, f1b_ref, f2w_ref, f2b_ref, og_ref, obeta_ref, ig_ref, ib_ref,
         o_ref, qkv_sc, ctx_sc, kt_sc, khat_sc, vhat_sc) = refs
    else:
        (x_ref, m_ref, qkvw_ref, qkvb_ref, ow_ref, ob_ref, sag_ref, sab_ref,
         f1w_ref, f1b_ref, f2w_ref, f2b_ref, og_ref, obeta_ref,
         o_ref, qkv_sc, ctx_sc, kt_sc, khat_sc, vhat_sc) = refs

    TB, S, H = x_ref.shape
    xv = x_ref[...].reshape(TB * S, H).astype(jnp.float32)
    if has_in_ln:
        xv = _ln(xv, ig_ref[...], ib_ref[...])
    x_bf = xv.astype(jnp.bfloat16)
    resid0 = x_bf.astype(jnp.float32)

    def dot_k2(lhs, w_ref_, col=None):
        hk = lhs.shape[1] // 2
        sl = slice(None) if col is None else col
        acc = jnp.dot(lhs[:, :hk], w_ref_[:hk, sl],
                      preferred_element_type=jnp.float32)
        return acc + jnp.dot(lhs[:, hk:], w_ref_[hk:, sl],
                             preferred_element_type=jnp.float32)

    qkv = dot_k2(x_bf, qkvw_ref)
    qkv_sc[...] = (qkv + qkvb_ref[...]).astype(jnp.bfloat16)

    NG = _NH // 4
    @pl.when(pl.program_id(0) == 0)
    def _zero_blockdiag():
        khat_sc[...] = jnp.zeros_like(khat_sc)
        vhat_sc[...] = jnp.zeros_like(vhat_sc)

    kt_sc[...] = qkv_sc[:, _H:2 * _H].T
    for h in range(_NH):
        j, i = divmod(h, 4)
        khat_sc[_DH * i:_DH * (i + 1),
                1024 * j + 256 * i:1024 * j + 256 * i + 256] = (
            kt_sc[_DH * h:_DH * (h + 1), :])
        vhat_sc[S * i:S * (i + 1),
                256 * j + _DH * i:256 * j + _DH * (i + 1)] = (
            qkv_sc[:, 2 * _H + _DH * h:2 * _H + _DH * (h + 1)])

    mrow = m_ref[0]
    keep = mrow > 0.0

    def scores(j):
        return jnp.dot(qkv_sc[:, 256 * j:256 * (j + 1)],
                       khat_sc[:, 1024 * j:1024 * (j + 1)],
                       preferred_element_type=jnp.float32)

    s4 = scores(0)
    for j in range(NG):
        s4_next = scores(j + 1) if j + 1 < NG else None
        ps, linvs = [], []
        for i in range(4):
            s = jnp.where(keep, s4[:, 256 * i:256 * (i + 1)],
                          jnp.float32(-1e9))
            m = jnp.max(s, axis=-1, keepdims=True)
            p = jnp.exp(s - m)
            l = jnp.sum(p, axis=-1, keepdims=True)
            ps.append(p.astype(jnp.bfloat16))
            linvs.append(jnp.broadcast_to(
                pl.reciprocal(l, approx=True), (S, _DH)))
        ctx4 = jnp.dot(jnp.concatenate(ps, axis=1),
                       vhat_sc[:, 256 * j:256 * (j + 1)],
                       preferred_element_type=jnp.float32)
        ctx4 = ctx4 * jnp.concatenate(linvs, axis=1)
        ctx_sc[:, 256 * j:256 * (j + 1)] = ctx4.astype(jnp.bfloat16)
        s4 = s4_next

    attn = dot_k2(ctx_sc[...], ow_ref)
    attn = (attn + ob_ref[...]).astype(jnp.bfloat16).astype(jnp.float32)
    x1 = _ln(attn + resid0, sag_ref[...], sab_ref[...]).astype(jnp.bfloat16)

    h2 = None
    h1 = dot_k2(x1, f1w_ref, col=slice(0, _FFN_CHUNK))
    for c in range(0, _FFN, _FFN_CHUNK):
        nc = c + _FFN_CHUNK
        h1_next = (dot_k2(x1, f1w_ref, col=slice(nc, nc + _FFN_CHUNK))
                   if nc < _FFN else None)
        g = _gelu(h1 + f1b_ref[:, c:c + _FFN_CHUNK]).astype(jnp.bfloat16)
        d = jnp.dot(g, f2w_ref[c:c + _FFN_CHUNK, :],
                    preferred_element_type=jnp.float32)
        h2 = d if h2 is None else h2 + d
        h1 = h1_next
    h2 = (h2 + f2b_ref[...]).astype(jnp.bfloat16).astype(jnp.float32)
    x2 = _ln(h2 + x1.astype(jnp.float32), og_ref[...], obeta_ref[...])
    o_ref[...] = x2.astype(jnp.bfloat16).reshape(TB, S, H)


def _encoder_layer(x, mask3, qkv_w, qkv_b, o_w, o_b, sa_g, sa_b,
                   f1w, f1b, f2w, f2b, out_g, out_b, in_ln=None, tb=1):
    B, S, H = x.shape

    def full(shape):
        return pl.BlockSpec(shape, lambda b: (0,) * len(shape))

    in_specs = [
        pl.BlockSpec((tb, S, H), lambda b: (b, 0, 0)),
        pl.BlockSpec((tb, 1, S), lambda b: (b, 0, 0)),
        full((H, 3 * H)), full((1, 3 * H)),
        full((H, H)), full((1, H)),
        full((1, H)), full((1, H)),
        full((H, _FFN)), full((1, _FFN)),
        full((_FFN, H)), full((1, H)),
        full((1, H)), full((1, H)),
    ]
    args = [x, mask3, qkv_w, qkv_b, o_w, o_b, sa_g, sa_b,
            f1w, f1b, f2w, f2b, out_g, out_b]
    if in_ln is not None:
        in_specs += [full((1, H)), full((1, H))]
        args += list(in_ln)

    return pl.pallas_call(
        functools.partial(_layer_body, in_ln is not None),
        grid=(B // tb,),
        in_specs=in_specs,
        out_specs=pl.BlockSpec((tb, S, H), lambda b: (b, 0, 0)),
        out_shape=jax.ShapeDtypeStruct((B, S, H), jnp.bfloat16),
        scratch_shapes=[pltpu.VMEM((tb * S, 3 * H), jnp.bfloat16),
                        pltpu.VMEM((tb * S, H), jnp.bfloat16),
                        pltpu.VMEM((H, S), jnp.bfloat16),
                        pltpu.VMEM((S, 3072), jnp.bfloat16),
                        pltpu.VMEM((4 * S, 768), jnp.bfloat16)],
        compiler_params=pltpu.CompilerParams(
            dimension_semantics=("parallel",),
            vmem_limit_bytes=_VMEM_LIMIT),
    )(*args)


def _row(v):
    return v.reshape(1, -1).astype(jnp.float32)


def kernel(input_ids, attention_mask, word_emb, pos_emb, emb_ln_g, emb_ln_b, cls_w, cls_b,
           layer0_q_w, layer0_q_b, layer0_k_w, layer0_k_b, layer0_v_w, layer0_v_b,
           layer0_o_w, layer0_o_b, layer0_sa_ln_g, layer0_sa_ln_b,
           layer0_ffn1_w, layer0_ffn1_b, layer0_ffn2_w, layer0_ffn2_b,
           layer0_out_ln_g, layer0_out_ln_b,
           layer1_q_w, layer1_q_b, layer1_k_w, layer1_k_b, layer1_v_w, layer1_v_b,
           layer1_o_w, layer1_o_b, layer1_sa_ln_g, layer1_sa_ln_b,
           layer1_ffn1_w, layer1_ffn1_b, layer1_ffn2_w, layer1_ffn2_b,
           layer1_out_ln_g, layer1_out_ln_b,
           layer2_q_w, layer2_q_b, layer2_k_w, layer2_k_b, layer2_v_w, layer2_v_b,
           layer2_o_w, layer2_o_b, layer2_sa_ln_g, layer2_sa_ln_b,
           layer2_ffn1_w, layer2_ffn1_b, layer2_ffn2_w, layer2_ffn2_b,
           layer2_out_ln_g, layer2_out_ln_b,
           layer3_q_w, layer3_q_b, layer3_k_w, layer3_k_b, layer3_v_w, layer3_v_b,
           layer3_o_w, layer3_o_b, layer3_sa_ln_g, layer3_sa_ln_b,
           layer3_ffn1_w, layer3_ffn1_b, layer3_ffn2_w, layer3_ffn2_b,
           layer3_out_ln_g, layer3_out_ln_b):
    B, S = input_ids.shape

    layers = [
        (layer0_q_w, layer0_q_b, layer0_k_w, layer0_k_b, layer0_v_w, layer0_v_b,
         layer0_o_w, layer0_o_b, layer0_sa_ln_g, layer0_sa_ln_b,
         layer0_ffn1_w, layer0_ffn1_b, layer0_ffn2_w, layer0_ffn2_b,
         layer0_out_ln_g, layer0_out_ln_b),
        (layer1_q_w, layer1_q_b, layer1_k_w, layer1_k_b, layer1_v_w, layer1_v_b,
         layer1_o_w, layer1_o_b, layer1_sa_ln_g, layer1_sa_ln_b,
         layer1_ffn1_w, layer1_ffn1_b, layer1_ffn2_w, layer1_ffn2_b,
         layer1_out_ln_g, layer1_out_ln_b),
        (layer2_q_w, layer2_q_b, layer2_k_w, layer2_k_b, layer2_v_w, layer2_v_b,
         layer2_o_w, layer2_o_b, layer2_sa_ln_g, layer2_sa_ln_b,
         layer2_ffn1_w, layer2_ffn1_b, layer2_ffn2_w, layer2_ffn2_b,
         layer2_out_ln_g, layer2_out_ln_b),
        (layer3_q_w, layer3_q_b, layer3_k_w, layer3_k_b, layer3_v_w, layer3_v_b,
         layer3_o_w, layer3_o_b, layer3_sa_ln_g, layer3_sa_ln_b,
         layer3_ffn1_w, layer3_ffn1_b, layer3_ffn2_w, layer3_ffn2_b,
         layer3_out_ln_g, layer3_out_ln_b),
    ]

    x = (word_emb[input_ids] + pos_emb[:S][None, :, :]).astype(jnp.bfloat16)
    mask3 = attention_mask.astype(jnp.float32).reshape(B, 1, S)

    for li, (q_w, q_b, k_w, k_b, v_w, v_b, o_w, o_b, sa_g, sa_b,
             f1w, f1b, f2w, f2b, out_g, out_b) in enumerate(layers):
        qkv_w = jnp.concatenate([q_w * _SCALE, k_w, v_w],
                                axis=1).astype(jnp.bfloat16)
        qkv_b = jnp.concatenate([q_b * _SCALE, k_b, v_b]).reshape(1, -1)
        in_ln = (_row(emb_ln_g), _row(emb_ln_b)) if li == 0 else None
        x = _encoder_layer(
            x, mask3, qkv_w, qkv_b,
            o_w.astype(jnp.bfloat16), _row(o_b), _row(sa_g), _row(sa_b),
            f1w.astype(jnp.bfloat16), _row(f1b),
            f2w.astype(jnp.bfloat16), _row(f2b), _row(out_g), _row(out_b),
            in_ln=in_ln)

    pooled = x[:, 0, :].astype(jnp.float32)
    return pooled @ cls_w + cls_b


# speedup vs baseline: 1.4966x; 1.0885x over previous
import functools
import math

import jax
import jax.numpy as jnp
from jax.experimental import pallas as pl
from jax.experimental.pallas import tpu as pltpu

_H = 768
_NH = 12
_DH = 64
_FFN = 3072
_FFN_CHUNK = 512
_SCALE = 1.0 / math.sqrt(_DH)
_VMEM_LIMIT = 48 * 1024 * 1024


def _ln(x, g, b, eps=1e-12):
    mu = jnp.mean(x, axis=-1, keepdims=True)
    d = x - mu
    var = jnp.mean(d * d, axis=-1, keepdims=True)
    return d * jax.lax.rsqrt(var + eps) * g + b


def _gelu(y):
    c = math.sqrt(2.0 / math.pi)
    return 0.5 * y * (1.0 + jnp.tanh(c * (y + 0.044715 * y * y * y)))


def _layer_body(has_in_ln, *refs):
    if has_in_ln:
        (x_ref, m_ref, qkvw_ref, qkvb_ref, ow_ref, ob_ref, sag_ref, sab_ref,
         f1w_ref, f1b_ref, f2w_ref, f2b_ref, og_ref, obeta_ref, ig_ref, ib_ref,
         o_ref, qkv_sc, ctx_sc, kt_sc, khat_sc, vhat_sc) = refs
    else:
        (x_ref, m_ref, qkvw_ref, qkvb_ref, ow_ref, ob_ref, sag_ref, sab_ref,
         f1w_ref, f1b_ref, f2w_ref, f2b_ref, og_ref, obeta_ref,
         o_ref, qkv_sc, ctx_sc, kt_sc, khat_sc, vhat_sc) = refs

    TB, S, H = x_ref.shape
    xv = x_ref[...].reshape(TB * S, H).astype(jnp.float32)
    if has_in_ln:
        xv = _ln(xv, ig_ref[...], ib_ref[...])
    x_bf = xv.astype(jnp.bfloat16)
    resid0 = x_bf.astype(jnp.float32)

    def dot_k2(lhs, w_ref_, col=None):
        hk = lhs.shape[1] // 2
        sl = slice(None) if col is None else col
        acc = jnp.dot(lhs[:, :hk], w_ref_[:hk, sl],
                      preferred_element_type=jnp.float32)
        return acc + jnp.dot(lhs[:, hk:], w_ref_[hk:, sl],
                             preferred_element_type=jnp.float32)

    qkv = dot_k2(x_bf, qkvw_ref)
    qkv_sc[...] = (qkv + qkvb_ref[...]).astype(jnp.bfloat16)

    NG = _NH // 4
    @pl.when(pl.program_id(0) == 0)
    def _zero_blockdiag():
        khat_sc[...] = jnp.zeros_like(khat_sc)
        vhat_sc[...] = jnp.zeros_like(vhat_sc)

    for be in range(TB):
        rows = slice(be * S, (be + 1) * S)
        kt_sc[...] = qkv_sc[rows, _H:2 * _H].T
        for h in range(_NH):
            j, i = divmod(h, 4)
            khat_sc[_DH * i:_DH * (i + 1),
                    1024 * j + 256 * i:1024 * j + 256 * i + 256] = (
                kt_sc[_DH * h:_DH * (h + 1), :])
            vhat_sc[S * i:S * (i + 1),
                    256 * j + _DH * i:256 * j + _DH * (i + 1)] = (
                qkv_sc[rows, 2 * _H + _DH * h:2 * _H + _DH * (h + 1)])

        keep = m_ref[be] > 0.0

        def scores(j, rows=rows):
            return jnp.dot(qkv_sc[rows, 256 * j:256 * (j + 1)],
                           khat_sc[:, 1024 * j:1024 * (j + 1)],
                           preferred_element_type=jnp.float32)

        s4 = scores(0)
        for j in range(NG):
            s4_next = scores(j + 1) if j + 1 < NG else None
            ps, linvs = [], []
            for i in range(4):
                s = jnp.where(keep, s4[:, 256 * i:256 * (i + 1)],
                              jnp.float32(-1e9))
                m = jnp.max(s, axis=-1, keepdims=True)
                p = jnp.exp(s - m)
                l = jnp.sum(p, axis=-1, keepdims=True)
                ps.append(p.astype(jnp.bfloat16))
                linvs.append(jnp.broadcast_to(
                    pl.reciprocal(l, approx=True), (S, _DH)))
            ctx4 = jnp.dot(jnp.concatenate(ps, axis=1),
                           vhat_sc[:, 256 * j:256 * (j + 1)],
                           preferred_element_type=jnp.float32)
            ctx4 = ctx4 * jnp.concatenate(linvs, axis=1)
            ctx_sc[rows, 256 * j:256 * (j + 1)] = ctx4.astype(jnp.bfloat16)
            s4 = s4_next

    attn = dot_k2(ctx_sc[...], ow_ref)
    attn = (attn + ob_ref[...]).astype(jnp.bfloat16).astype(jnp.float32)
    x1 = _ln(attn + resid0, sag_ref[...], sab_ref[...]).astype(jnp.bfloat16)

    h2 = None
    h1 = dot_k2(x1, f1w_ref, col=slice(0, _FFN_CHUNK))
    for c in range(0, _FFN, _FFN_CHUNK):
        nc = c + _FFN_CHUNK
        h1_next = (dot_k2(x1, f1w_ref, col=slice(nc, nc + _FFN_CHUNK))
                   if nc < _FFN else None)
        g = _gelu(h1 + f1b_ref[:, c:c + _FFN_CHUNK]).astype(jnp.bfloat16)
        d = jnp.dot(g, f2w_ref[c:c + _FFN_CHUNK, :],
                    preferred_element_type=jnp.float32)
        h2 = d if h2 is None else h2 + d
        h1 = h1_next
    h2 = (h2 + f2b_ref[...]).astype(jnp.bfloat16).astype(jnp.float32)
    x2 = _ln(h2 + x1.astype(jnp.float32), og_ref[...], obeta_ref[...])
    o_ref[...] = x2.astype(jnp.bfloat16).reshape(TB, S, H)


def _encoder_layer(x, mask3, qkv_w, qkv_b, o_w, o_b, sa_g, sa_b,
                   f1w, f1b, f2w, f2b, out_g, out_b, in_ln=None, tb=2):
    B, S, H = x.shape

    def full(shape):
        return pl.BlockSpec(shape, lambda b: (0,) * len(shape))

    in_specs = [
        pl.BlockSpec((tb, S, H), lambda b: (b, 0, 0)),
        pl.BlockSpec((tb, 1, S), lambda b: (b, 0, 0)),
        full((H, 3 * H)), full((1, 3 * H)),
        full((H, H)), full((1, H)),
        full((1, H)), full((1, H)),
        full((H, _FFN)), full((1, _FFN)),
        full((_FFN, H)), full((1, H)),
        full((1, H)), full((1, H)),
    ]
    args = [x, mask3, qkv_w, qkv_b, o_w, o_b, sa_g, sa_b,
            f1w, f1b, f2w, f2b, out_g, out_b]
    if in_ln is not None:
        in_specs += [full((1, H)), full((1, H))]
        args += list(in_ln)

    return pl.pallas_call(
        functools.partial(_layer_body, in_ln is not None),
        grid=(B // tb,),
        in_specs=in_specs,
        out_specs=pl.BlockSpec((tb, S, H), lambda b: (b, 0, 0)),
        out_shape=jax.ShapeDtypeStruct((B, S, H), jnp.bfloat16),
        scratch_shapes=[pltpu.VMEM((tb * S, 3 * H), jnp.bfloat16),
                        pltpu.VMEM((tb * S, H), jnp.bfloat16),
                        pltpu.VMEM((H, S), jnp.bfloat16),
                        pltpu.VMEM((S, 3072), jnp.bfloat16),
                        pltpu.VMEM((4 * S, 768), jnp.bfloat16)],
        compiler_params=pltpu.CompilerParams(
            dimension_semantics=("parallel",),
            vmem_limit_bytes=_VMEM_LIMIT),
    )(*args)


def _row(v):
    return v.reshape(1, -1).astype(jnp.float32)


def kernel(input_ids, attention_mask, word_emb, pos_emb, emb_ln_g, emb_ln_b, cls_w, cls_b,
           layer0_q_w, layer0_q_b, layer0_k_w, layer0_k_b, layer0_v_w, layer0_v_b,
           layer0_o_w, layer0_o_b, layer0_sa_ln_g, layer0_sa_ln_b,
           layer0_ffn1_w, layer0_ffn1_b, layer0_ffn2_w, layer0_ffn2_b,
           layer0_out_ln_g, layer0_out_ln_b,
           layer1_q_w, layer1_q_b, layer1_k_w, layer1_k_b, layer1_v_w, layer1_v_b,
           layer1_o_w, layer1_o_b, layer1_sa_ln_g, layer1_sa_ln_b,
           layer1_ffn1_w, layer1_ffn1_b, layer1_ffn2_w, layer1_ffn2_b,
           layer1_out_ln_g, layer1_out_ln_b,
           layer2_q_w, layer2_q_b, layer2_k_w, layer2_k_b, layer2_v_w, layer2_v_b,
           layer2_o_w, layer2_o_b, layer2_sa_ln_g, layer2_sa_ln_b,
           layer2_ffn1_w, layer2_ffn1_b, layer2_ffn2_w, layer2_ffn2_b,
           layer2_out_ln_g, layer2_out_ln_b,
           layer3_q_w, layer3_q_b, layer3_k_w, layer3_k_b, layer3_v_w, layer3_v_b,
           layer3_o_w, layer3_o_b, layer3_sa_ln_g, layer3_sa_ln_b,
           layer3_ffn1_w, layer3_ffn1_b, layer3_ffn2_w, layer3_ffn2_b,
           layer3_out_ln_g, layer3_out_ln_b):
    B, S = input_ids.shape

    layers = [
        (layer0_q_w, layer0_q_b, layer0_k_w, layer0_k_b, layer0_v_w, layer0_v_b,
         layer0_o_w, layer0_o_b, layer0_sa_ln_g, layer0_sa_ln_b,
         layer0_ffn1_w, layer0_ffn1_b, layer0_ffn2_w, layer0_ffn2_b,
         layer0_out_ln_g, layer0_out_ln_b),
        (layer1_q_w, layer1_q_b, layer1_k_w, layer1_k_b, layer1_v_w, layer1_v_b,
         layer1_o_w, layer1_o_b, layer1_sa_ln_g, layer1_sa_ln_b,
         layer1_ffn1_w, layer1_ffn1_b, layer1_ffn2_w, layer1_ffn2_b,
         layer1_out_ln_g, layer1_out_ln_b),
        (layer2_q_w, layer2_q_b, layer2_k_w, layer2_k_b, layer2_v_w, layer2_v_b,
         layer2_o_w, layer2_o_b, layer2_sa_ln_g, layer2_sa_ln_b,
         layer2_ffn1_w, layer2_ffn1_b, layer2_ffn2_w, layer2_ffn2_b,
         layer2_out_ln_g, layer2_out_ln_b),
        (layer3_q_w, layer3_q_b, layer3_k_w, layer3_k_b, layer3_v_w, layer3_v_b,
         layer3_o_w, layer3_o_b, layer3_sa_ln_g, layer3_sa_ln_b,
         layer3_ffn1_w, layer3_ffn1_b, layer3_ffn2_w, layer3_ffn2_b,
         layer3_out_ln_g, layer3_out_ln_b),
    ]

    x = (word_emb[input_ids] + pos_emb[:S][None, :, :]).astype(jnp.bfloat16)
    mask3 = attention_mask.astype(jnp.float32).reshape(B, 1, S)

    for li, (q_w, q_b, k_w, k_b, v_w, v_b, o_w, o_b, sa_g, sa_b,
             f1w, f1b, f2w, f2b, out_g, out_b) in enumerate(layers):
        qkv_w = jnp.concatenate([q_w * _SCALE, k_w, v_w],
                                axis=1).astype(jnp.bfloat16)
        qkv_b = jnp.concatenate([q_b * _SCALE, k_b, v_b]).reshape(1, -1)
        in_ln = (_row(emb_ln_g), _row(emb_ln_b)) if li == 0 else None
        x = _encoder_layer(
            x, mask3, qkv_w, qkv_b,
            o_w.astype(jnp.bfloat16), _row(o_b), _row(sa_g), _row(sa_b),
            f1w.astype(jnp.bfloat16), _row(f1b),
            f2w.astype(jnp.bfloat16), _row(f2b), _row(out_g), _row(out_b),
            in_ln=in_ln)

    pooled = x[:, 0, :].astype(jnp.float32)
    return pooled @ cls_w + cls_b
